# Initial kernel scaffold; baseline (speedup 1.0000x reference)
import functools

import jax
import jax.numpy as jnp
from jax import lax
from jax.experimental import pallas as pl
from jax.experimental.pallas import tpu as pltpu
from jax.experimental.pallas import tpu_sc as plsc

N = 100000
E = 1600000
HID = 32

NC = 2
NS = 16
NW = NC * NS
CH = 1024
EPW = 50176
E_PAD = NW * EPW
NCH_G = EPW // CH
PER_TILE = E_PAD // NS
NCH_S = PER_TILE // CH
HALF = N // NC
ZROWS = 3128
ACC_ROWS = NS * ZROWS
CPR = N // NC // NS

BE = 8192
BN = 2000

_mesh = plsc.VectorSubcoreMesh(core_axis_name="c", subcore_axis_name="s")


@functools.partial(
    pl.kernel,
    mesh=_mesh,
    out_type=(
        jax.ShapeDtypeStruct((E_PAD, HID), jnp.float32),
        jax.ShapeDtypeStruct((E_PAD, HID), jnp.float32),
    ),
    scratch_types=[
        pltpu.VMEM((8, 128), jnp.int32),
        pltpu.VMEM((8, 128), jnp.int32),
        pltpu.VMEM((CH, HID), jnp.float32),
        pltpu.VMEM((CH, HID), jnp.float32),
        pltpu.SemaphoreType.DMA,
    ],
)
def _sc_gather(h_hbm, dst2_hbm, src2_hbm, ga_hbm, gb_hbm,
               idxd, idxs, ga_v, gb_v, sem):
    c = lax.axis_index("c")
    s = lax.axis_index("s")
    wid = s * NC + c
    base = wid * EPW

    def body(i, carry):
        e0 = base + i * CH
        r0 = e0 // 128
        pltpu.sync_copy(dst2_hbm.at[pl.ds(r0, 8)], idxd)
        pltpu.sync_copy(src2_hbm.at[pl.ds(r0, 8)], idxs)
        cps = []
        for j in range(8):
            cps.append(pltpu.async_copy(
                h_hbm.at[idxd.at[j]], ga_v.at[pl.ds(j * 128, 128)], sem))
        for j in range(8):
            cps.append(pltpu.async_copy(
                h_hbm.at[idxs.at[j]], gb_v.at[pl.ds(j * 128, 128)], sem))
        for cp in cps:
            cp.wait()
        pltpu.sync_copy(ga_v, ga_hbm.at[pl.ds(e0, CH)])
        pltpu.sync_copy(gb_v, gb_hbm.at[pl.ds(e0, CH)])
        return carry

    lax.fori_loop(0, NCH_G, body, 0)


@functools.partial(
    pl.kernel,
    mesh=_mesh,
    out_type=jax.ShapeDtypeStruct((N, HID), jnp.float32),
    scratch_types=[
        pltpu.VMEM((8, 128), jnp.int32),
        pltpu.VMEM((8, 128), jnp.int32),
        pltpu.VMEM((CH, HID), jnp.float32),
        pltpu.VMEM_SHARED((ACC_ROWS, HID), jnp.float32),
        pltpu.SemaphoreType.DMA,
    ],
)
def _sc_scatter(upd_hbm, src2_hbm, zeros_hbm, agg_hbm,
                idxs, idxl, rows_v, acc, sem):
    c = lax.axis_index("c")
    s = lax.axis_index("s")
    nbase = c * HALF
    pltpu.sync_copy(zeros_hbm, acc.at[pl.ds(s * ZROWS, ZROWS)])
    plsc.subcore_barrier()

    t0 = s * PER_TILE

    def body(i, carry):
        e0 = t0 + i * CH
        r0 = e0 // 128
        pltpu.sync_copy(src2_hbm.at[pl.ds(r0, 8)], idxs)
        pltpu.sync_copy(upd_hbm.at[pl.ds(e0, CH)], rows_v)
        for j in range(8):
            for k in range(8):
                v = idxs[j, pl.ds(k * 16, 16)]
                loc = v - nbase
                ok = (loc >= 0) & (loc < HALF)
                idxl[j, pl.ds(k * 16, 16)] = jnp.where(ok, loc, HALF)
        for j in range(8):
            pltpu.sync_copy(rows_v.at[pl.ds(j * 128, 128)],
                            acc.at[idxl.at[j]], add=True)
        return carry

    lax.fori_loop(0, NCH_S, body, 0)
    plsc.subcore_barrier()
    pltpu.sync_copy(acc.at[pl.ds(s * CPR, CPR)],
                    agg_hbm.at[pl.ds(nbase + s * CPR, CPR)])


def _ln(u, g, beta):
    mu = jnp.mean(u, axis=-1, keepdims=True)
    var = jnp.mean((u - mu) * (u - mu), axis=-1, keepdims=True)
    return (u - mu) * lax.rsqrt(var + 1e-5) * g + beta


def _full(a):
    return pl.BlockSpec(a.shape, lambda i: tuple(0 for _ in a.shape))


def _rows(block, width):
    return pl.BlockSpec((block, width), lambda i: (i, 0))


def _enc_body(x_ref, mean_ref, std_ref, W1, b1, W2, b2, g, beta, out_ref):
    xn = (x_ref[...] - mean_ref[...]) / std_ref[...]
    t = jnp.maximum(jnp.dot(xn, W1[...], preferred_element_type=jnp.float32)
                    + b1[...], 0.0)
    u = jnp.dot(t, W2[...], preferred_element_type=jnp.float32) + b2[...]
    out_ref[...] = _ln(u, g[...], beta[...])


def _tc_encoder(arr, mean, std, p_mlp, block):
    n = arr.shape[0]
    args = (arr, mean, std, p_mlp["W1"], p_mlp["b1"], p_mlp["W2"],
            p_mlp["b2"], p_mlp["g"], p_mlp["beta"])
    return pl.pallas_call(
        _enc_body,
        grid=(n // block,),
        in_specs=[_rows(block, arr.shape[1])] + [_full(a) for a in args[1:]],
        out_specs=_rows(block, HID),
        out_shape=jax.ShapeDtypeStruct((n, HID), jnp.float32),
    )(*args)


def _edge_body(ga, gb, e, W1a, W1b, W1c, b1, W2, b2, g, beta, out_ref):
    t = (jnp.dot(ga[...], W1a[...], preferred_element_type=jnp.float32)
         + jnp.dot(gb[...], W1b[...], preferred_element_type=jnp.float32)
         + jnp.dot(e[...], W1c[...], preferred_element_type=jnp.float32)
         + b1[...])
    t = jnp.maximum(t, 0.0)
    u = jnp.dot(t, W2[...], preferred_element_type=jnp.float32) + b2[...]
    out_ref[...] = _ln(u, g[...], beta[...]) + e[...]


def _tc_edge(ga, gb, e, w):
    args = (ga, gb, e, w["W1a"], w["W1b"], w["W1c"], w["b1"], w["W2"],
            w["b2"], w["g"], w["beta"])
    return pl.pallas_call(
        _edge_body,
        grid=(E_PAD // BE,),
        in_specs=[_rows(BE, HID)] * 3 + [_full(a) for a in args[3:]],
        out_specs=_rows(BE, HID),
        out_shape=jax.ShapeDtypeStruct((E_PAD, HID), jnp.float32),
    )(*args)


def _node_body(h, agg, W1a, W1b, b1, W2, b2, g, beta, out_ref):
    t = (jnp.dot(h[...], W1a[...], preferred_element_type=jnp.float32)
         + jnp.dot(agg[...], W1b[...], preferred_element_type=jnp.float32)
         + b1[...])
    t = jnp.maximum(t, 0.0)
    u = jnp.dot(t, W2[...], preferred_element_type=jnp.float32) + b2[...]
    out_ref[...] = h[...] + _ln(u, g[...], beta[...])


def _tc_node(h, agg, w):
    args = (h, agg, w["W1a"], w["W1b"], w["b1"], w["W2"], w["b2"],
            w["g"], w["beta"])
    return pl.pallas_call(
        _node_body,
        grid=(N // BN,),
        in_specs=[_rows(BN, HID)] * 2 + [_full(a) for a in args[2:]],
        out_specs=_rows(BN, HID),
        out_shape=jax.ShapeDtypeStruct((N, HID), jnp.float32),
    )(*args)


def _dec_body(h, W1, b1, W2, b2, out_ref):
    t = jnp.maximum(jnp.dot(h[...], W1[...],
                            preferred_element_type=jnp.float32) + b1[...], 0.0)
    out_ref[...] = jnp.dot(t, W2[...],
                           preferred_element_type=jnp.float32) + b2[...]


def _tc_dec(h, d):
    args = (h, d["W1"], d["b1"], d["W2"], d["b2"])
    return pl.pallas_call(
        _dec_body,
        grid=(N // BN,),
        in_specs=[_rows(BN, HID)] + [_full(a) for a in args[1:]],
        out_specs=pl.BlockSpec((BN, 2), lambda i: (i, 0)),
        out_shape=jax.ShapeDtypeStruct((N, 2), jnp.float32),
    )(*args)


def _prep_mlp(pr):
    return {"W1": pr["W1"], "b1": pr["b1"].reshape(1, -1),
            "W2": pr["W2"], "b2": pr["b2"].reshape(1, -1),
            "g": pr["g"].reshape(1, -1), "beta": pr["beta"].reshape(1, -1)}


def kernel(x, edge_index, edge_attr, p, mean_vec_x, std_vec_x,
           mean_vec_edge, std_vec_edge, params):
    pad = E_PAD - E
    src = edge_index[0]
    dst = edge_index[1]
    src_g2 = jnp.concatenate([src, jnp.zeros((pad,), jnp.int32)]
                             ).reshape(E_PAD // 128, 128)
    dst_g2 = jnp.concatenate([dst, jnp.zeros((pad,), jnp.int32)]
                             ).reshape(E_PAD // 128, 128)
    src_s2 = jnp.concatenate([src, jnp.full((pad,), N, jnp.int32)]
                             ).reshape(E_PAD // 128, 128)
    ea_pad = jnp.concatenate(
        [edge_attr, jnp.zeros((pad, edge_attr.shape[1]), jnp.float32)])
    zeros_tile = jnp.zeros((ZROWS, HID), jnp.float32)

    h = _tc_encoder(x, mean_vec_x.reshape(1, -1), std_vec_x.reshape(1, -1),
                    _prep_mlp(params["node_enc"]), BN)
    e = _tc_encoder(ea_pad, mean_vec_edge.reshape(1, -1),
                    std_vec_edge.reshape(1, -1),
                    _prep_mlp(params["edge_enc"]), BE)

    for lp in params["layers"]:
        em = _prep_mlp(lp["edge_mlp"])
        ew = {"W1a": em["W1"][:HID], "W1b": em["W1"][HID:2 * HID],
              "W1c": em["W1"][2 * HID:], "b1": em["b1"], "W2": em["W2"],
              "b2": em["b2"], "g": em["g"], "beta": em["beta"]}
        nm = _prep_mlp(lp["node_mlp"])
        nw = {"W1a": nm["W1"][:HID], "W1b": nm["W1"][HID:], "b1": nm["b1"],
              "W2": nm["W2"], "b2": nm["b2"], "g": nm["g"],
              "beta": nm["beta"]}
        ga, gb = _sc_gather(h, dst_g2, src_g2)
        upd = _tc_edge(ga, gb, e, ew)
        agg = _sc_scatter(upd, src_s2, zeros_tile)
        h = _tc_node(h, agg, nw)
        e = upd

    d = params["dec"]
    return _tc_dec(h, {"W1": d["W1"], "b1": d["b1"].reshape(1, -1),
                       "W2": d["W2"], "b2": d["b2"].reshape(1, -1)})


# speedup vs baseline: 2.0487x; 2.0487x over previous
import functools

import jax
import jax.numpy as jnp
from jax import lax
from jax.experimental import pallas as pl
from jax.experimental.pallas import tpu as pltpu
from jax.experimental.pallas import tpu_sc as plsc

N = 100000
E = 1600000
HID = 32

NC = 2
NS = 16
NW = NC * NS
CH = 1024
EPW = 50176
E_PAD = NW * EPW
NCH_G = EPW // CH
CHS = 512
PER_TILE = E_PAD // NS
NCH_S = PER_TILE // CHS
HALF = N // NC
ZROWS = 3128
ACC_ROWS = NS * ZROWS
CPR = N // NC // NS

BE = 8192
BN = 2000

_mesh = plsc.VectorSubcoreMesh(core_axis_name="c", subcore_axis_name="s")


@functools.partial(
    pl.kernel,
    mesh=_mesh,
    compiler_params=pltpu.CompilerParams(use_tc_tiling_on_sc=False),
    out_type=(
        jax.ShapeDtypeStruct((E_PAD, HID), jnp.float32),
        jax.ShapeDtypeStruct((E_PAD, HID), jnp.float32),
    ),
    scratch_types=[
        pltpu.VMEM((8, 128), jnp.int32),
        pltpu.VMEM((8, 128), jnp.int32),
        pltpu.VMEM((CH, HID), jnp.float32),
        pltpu.VMEM((CH, HID), jnp.float32),
        pltpu.SemaphoreType.DMA,
    ],
)
def _sc_gather(h_hbm, dst2_hbm, src2_hbm, ga_hbm, gb_hbm,
               idxd, idxs, ga_v, gb_v, sem):
    c = lax.axis_index("c")
    s = lax.axis_index("s")
    wid = s * NC + c
    base = wid * EPW

    def body(i, carry):
        e0 = base + i * CH
        r0 = pl.multiple_of(e0 // 128, 8)
        pltpu.sync_copy(dst2_hbm.at[pl.ds(r0, 8)], idxd)
        pltpu.sync_copy(src2_hbm.at[pl.ds(r0, 8)], idxs)
        cps = []
        for j in range(8):
            cps.append(pltpu.async_copy(
                h_hbm.at[idxd.at[j]], ga_v.at[pl.ds(j * 128, 128)], sem))
        for j in range(8):
            cps.append(pltpu.async_copy(
                h_hbm.at[idxs.at[j]], gb_v.at[pl.ds(j * 128, 128)], sem))
        for cp in cps:
            cp.wait()
        pltpu.sync_copy(ga_v, ga_hbm.at[pl.ds(e0, CH)])
        pltpu.sync_copy(gb_v, gb_hbm.at[pl.ds(e0, CH)])
        return carry

    lax.fori_loop(0, NCH_G, body, 0)


@functools.partial(
    pl.kernel,
    mesh=_mesh,
    compiler_params=pltpu.CompilerParams(use_tc_tiling_on_sc=False),
    out_type=jax.ShapeDtypeStruct((N, HID), jnp.float32),
    scratch_types=[
        pltpu.VMEM((4, 128), jnp.int32),
        pltpu.VMEM((4, 128), jnp.int32),
        pltpu.VMEM((CHS, HID), jnp.float32),
        pltpu.VMEM_SHARED((ACC_ROWS, HID), jnp.float32),
        pltpu.SemaphoreType.DMA,
    ],
)
def _sc_scatter(upd_hbm, src2_hbm, zeros_hbm, agg_hbm,
                idxs, idxl, rows_v, acc, sem):
    c = lax.axis_index("c")
    s = lax.axis_index("s")
    nbase = c * HALF
    pltpu.sync_copy(zeros_hbm, acc.at[pl.ds(s * ZROWS, ZROWS)])
    plsc.subcore_barrier()

    t0 = s * PER_TILE

    def body(i, carry):
        e0 = t0 + i * CHS
        r0 = pl.multiple_of(e0 // 128, 4)
        pltpu.sync_copy(src2_hbm.at[pl.ds(r0, 4)], idxs)
        pltpu.sync_copy(upd_hbm.at[pl.ds(e0, CHS)], rows_v)
        for j in range(4):
            for k in range(8):
                v = idxs[j, pl.ds(k * 16, 16)]
                loc = v - nbase
                ok = (loc >= 0) & (loc < HALF)
                idxl[j, pl.ds(k * 16, 16)] = jnp.where(ok, loc, HALF)
        for j in range(4):
            pltpu.sync_copy(rows_v.at[pl.ds(j * 128, 128)],
                            acc.at[idxl.at[j]], add=True)
        return carry

    lax.fori_loop(0, NCH_S, body, 0)
    plsc.subcore_barrier()
    pltpu.sync_copy(acc.at[pl.ds(s * CPR, CPR)],
                    agg_hbm.at[pl.ds(nbase + s * CPR, CPR)])


def _ln(u, g, beta):
    mu = jnp.mean(u, axis=-1, keepdims=True)
    var = jnp.mean((u - mu) * (u - mu), axis=-1, keepdims=True)
    return (u - mu) * lax.rsqrt(var + 1e-5) * g + beta


def _full(a):
    return pl.BlockSpec(a.shape, lambda i: tuple(0 for _ in a.shape))


def _rows(block, width):
    return pl.BlockSpec((block, width), lambda i: (i, 0))


def _enc_body(x_ref, mean_ref, std_ref, W1, b1, W2, b2, g, beta, out_ref):
    xn = (x_ref[...] - mean_ref[...]) / std_ref[...]
    t = jnp.maximum(jnp.dot(xn, W1[...], preferred_element_type=jnp.float32)
                    + b1[...], 0.0)
    u = jnp.dot(t, W2[...], preferred_element_type=jnp.float32) + b2[...]
    out_ref[...] = _ln(u, g[...], beta[...])


def _tc_encoder(arr, mean, std, p_mlp, block):
    n = arr.shape[0]
    args = (arr, mean, std, p_mlp["W1"], p_mlp["b1"], p_mlp["W2"],
            p_mlp["b2"], p_mlp["g"], p_mlp["beta"])
    return pl.pallas_call(
        _enc_body,
        grid=(n // block,),
        in_specs=[_rows(block, arr.shape[1])] + [_full(a) for a in args[1:]],
        out_specs=_rows(block, HID),
        out_shape=jax.ShapeDtypeStruct((n, HID), jnp.float32),
    )(*args)


def _edge_body(ga, gb, e, W1a, W1b, W1c, b1, W2, b2, g, beta, out_ref):
    t = (jnp.dot(ga[...], W1a[...], preferred_element_type=jnp.float32)
         + jnp.dot(gb[...], W1b[...], preferred_element_type=jnp.float32)
         + jnp.dot(e[...], W1c[...], preferred_element_type=jnp.float32)
         + b1[...])
    t = jnp.maximum(t, 0.0)
    u = jnp.dot(t, W2[...], preferred_element_type=jnp.float32) + b2[...]
    out_ref[...] = _ln(u, g[...], beta[...]) + e[...]


def _tc_edge(ga, gb, e, w):
    args = (ga, gb, e, w["W1a"], w["W1b"], w["W1c"], w["b1"], w["W2"],
            w["b2"], w["g"], w["beta"])
    return pl.pallas_call(
        _edge_body,
        grid=(E_PAD // BE,),
        in_specs=[_rows(BE, HID)] * 3 + [_full(a) for a in args[3:]],
        out_specs=_rows(BE, HID),
        out_shape=jax.ShapeDtypeStruct((E_PAD, HID), jnp.float32),
    )(*args)


def _node_body(h, agg, W1a, W1b, b1, W2, b2, g, beta, out_ref):
    t = (jnp.dot(h[...], W1a[...], preferred_element_type=jnp.float32)
         + jnp.dot(agg[...], W1b[...], preferred_element_type=jnp.float32)
         + b1[...])
    t = jnp.maximum(t, 0.0)
    u = jnp.dot(t, W2[...], preferred_element_type=jnp.float32) + b2[...]
    out_ref[...] = h[...] + _ln(u, g[...], beta[...])


def _tc_node(h, agg, w):
    args = (h, agg, w["W1a"], w["W1b"], w["b1"], w["W2"], w["b2"],
            w["g"], w["beta"])
    return pl.pallas_call(
        _node_body,
        grid=(N // BN,),
        in_specs=[_rows(BN, HID)] * 2 + [_full(a) for a in args[2:]],
        out_specs=_rows(BN, HID),
        out_shape=jax.ShapeDtypeStruct((N, HID), jnp.float32),
    )(*args)


def _dec_body(h, W1, b1, W2, b2, out_ref):
    t = jnp.maximum(jnp.dot(h[...], W1[...],
                            preferred_element_type=jnp.float32) + b1[...], 0.0)
    out_ref[...] = jnp.dot(t, W2[...],
                           preferred_element_type=jnp.float32) + b2[...]


def _tc_dec(h, d):
    args = (h, d["W1"], d["b1"], d["W2"], d["b2"])
    return pl.pallas_call(
        _dec_body,
        grid=(N // BN,),
        in_specs=[_rows(BN, HID)] + [_full(a) for a in args[1:]],
        out_specs=pl.BlockSpec((BN, 2), lambda i: (i, 0)),
        out_shape=jax.ShapeDtypeStruct((N, 2), jnp.float32),
    )(*args)


def _prep_mlp(pr):
    return {"W1": pr["W1"], "b1": pr["b1"].reshape(1, -1),
            "W2": pr["W2"], "b2": pr["b2"].reshape(1, -1),
            "g": pr["g"].reshape(1, -1), "beta": pr["beta"].reshape(1, -1)}


def kernel(x, edge_index, edge_attr, p, mean_vec_x, std_vec_x,
           mean_vec_edge, std_vec_edge, params):
    pad = E_PAD - E
    src = edge_index[0]
    dst = edge_index[1]
    src_g2 = jnp.concatenate([src, jnp.zeros((pad,), jnp.int32)]
                             ).reshape(E_PAD // 128, 128)
    dst_g2 = jnp.concatenate([dst, jnp.zeros((pad,), jnp.int32)]
                             ).reshape(E_PAD // 128, 128)
    src_s2 = jnp.concatenate([src, jnp.full((pad,), N, jnp.int32)]
                             ).reshape(E_PAD // 128, 128)
    ea_pad = jnp.concatenate(
        [edge_attr, jnp.zeros((pad, edge_attr.shape[1]), jnp.float32)])
    zeros_tile = jnp.zeros((ZROWS, HID), jnp.float32)

    h = _tc_encoder(x, mean_vec_x.reshape(1, -1), std_vec_x.reshape(1, -1),
                    _prep_mlp(params["node_enc"]), BN)
    e = _tc_encoder(ea_pad, mean_vec_edge.reshape(1, -1),
                    std_vec_edge.reshape(1, -1),
                    _prep_mlp(params["edge_enc"]), BE)

    for lp in params["layers"]:
        em = _prep_mlp(lp["edge_mlp"])
        ew = {"W1a": em["W1"][:HID], "W1b": em["W1"][HID:2 * HID],
              "W1c": em["W1"][2 * HID:], "b1": em["b1"], "W2": em["W2"],
              "b2": em["b2"], "g": em["g"], "beta": em["beta"]}
        nm = _prep_mlp(lp["node_mlp"])
        nw = {"W1a": nm["W1"][:HID], "W1b": nm["W1"][HID:], "b1": nm["b1"],
              "W2": nm["W2"], "b2": nm["b2"], "g": nm["g"],
              "beta": nm["beta"]}
        ga, gb = _sc_gather(h, dst_g2, src_g2)
        upd = _tc_edge(ga, gb, e, ew)
        agg = _sc_scatter(upd, src_s2, zeros_tile)
        h = _tc_node(h, agg, nw)
        e = upd

    d = params["dec"]
    return _tc_dec(h, {"W1": d["W1"], "b1": d["b1"].reshape(1, -1),
                       "W2": d["W2"], "b2": d["b2"].reshape(1, -1)})


# speedup vs baseline: 3.6113x; 1.7627x over previous
import functools

import jax
import jax.numpy as jnp
from jax import lax
from jax.experimental import pallas as pl
from jax.experimental.pallas import tpu as pltpu
from jax.experimental.pallas import tpu_sc as plsc

N = 100000
E = 1600000
HID = 32

NC = 2
NS = 16
NW = NC * NS
CH = 1024
EPW = 50176
E_PAD = NW * EPW
NCH_G = EPW // CH
CHS = 512
PER_TILE = E_PAD // NS
NCH_S = PER_TILE // CHS
HALF = N // NC
ZROWS = 3128
ACC_ROWS = NS * ZROWS
CPR = N // NC // NS

E4 = E_PAD // 4
N4 = N // 4
BE4 = 2048
BN4 = 5000

_mesh = plsc.VectorSubcoreMesh(core_axis_name="c", subcore_axis_name="s")


@functools.partial(
    pl.kernel,
    mesh=_mesh,
    compiler_params=pltpu.CompilerParams(use_tc_tiling_on_sc=False),
    out_type=(
        jax.ShapeDtypeStruct((E_PAD, HID), jnp.float32),
        jax.ShapeDtypeStruct((E_PAD, HID), jnp.float32),
    ),
    scratch_types=[
        pltpu.VMEM((8, 128), jnp.int32),
        pltpu.VMEM((8, 128), jnp.int32),
        pltpu.VMEM((CH, HID), jnp.float32),
        pltpu.VMEM((CH, HID), jnp.float32),
        pltpu.SemaphoreType.DMA,
    ],
)
def _sc_gather(h_hbm, dst2_hbm, src2_hbm, ga_hbm, gb_hbm,
               idxd, idxs, ga_v, gb_v, sem):
    c = lax.axis_index("c")
    s = lax.axis_index("s")
    wid = s * NC + c
    base = wid * EPW

    def body(i, carry):
        e0 = base + i * CH
        r0 = pl.multiple_of(e0 // 128, 8)
        pltpu.sync_copy(dst2_hbm.at[pl.ds(r0, 8)], idxd)
        pltpu.sync_copy(src2_hbm.at[pl.ds(r0, 8)], idxs)
        cps = []
        for j in range(8):
            cps.append(pltpu.async_copy(
                h_hbm.at[idxd.at[j]], ga_v.at[pl.ds(j * 128, 128)], sem))
        for j in range(8):
            cps.append(pltpu.async_copy(
                h_hbm.at[idxs.at[j]], gb_v.at[pl.ds(j * 128, 128)], sem))
        for cp in cps:
            cp.wait()
        pltpu.sync_copy(ga_v, ga_hbm.at[pl.ds(e0, CH)])
        pltpu.sync_copy(gb_v, gb_hbm.at[pl.ds(e0, CH)])
        return carry

    lax.fori_loop(0, NCH_G, body, 0)


@functools.partial(
    pl.kernel,
    mesh=_mesh,
    compiler_params=pltpu.CompilerParams(use_tc_tiling_on_sc=False),
    out_type=jax.ShapeDtypeStruct((N, HID), jnp.float32),
    scratch_types=[
        pltpu.VMEM((4, 128), jnp.int32),
        pltpu.VMEM((4, 128), jnp.int32),
        pltpu.VMEM((CHS, HID), jnp.float32),
        pltpu.VMEM_SHARED((ACC_ROWS, HID), jnp.float32),
        pltpu.SemaphoreType.DMA,
    ],
)
def _sc_scatter(upd_hbm, src2_hbm, zeros_hbm, agg_hbm,
                idxs, idxl, rows_v, acc, sem):
    c = lax.axis_index("c")
    s = lax.axis_index("s")
    nbase = c * HALF
    pltpu.sync_copy(zeros_hbm, acc.at[pl.ds(s * ZROWS, ZROWS)])
    plsc.subcore_barrier()

    t0 = s * PER_TILE

    def body(i, carry):
        e0 = t0 + i * CHS
        r0 = pl.multiple_of(e0 // 128, 4)
        pltpu.sync_copy(src2_hbm.at[pl.ds(r0, 4)], idxs)
        pltpu.sync_copy(upd_hbm.at[pl.ds(e0, CHS)], rows_v)
        for j in range(4):
            for k in range(8):
                v = idxs[j, pl.ds(k * 16, 16)]
                loc = v - nbase
                ok = (loc >= 0) & (loc < HALF)
                idxl[j, pl.ds(k * 16, 16)] = jnp.where(ok, loc, HALF)
        for j in range(4):
            pltpu.sync_copy(rows_v.at[pl.ds(j * 128, 128)],
                            acc.at[idxl.at[j]], add=True)
        return carry

    lax.fori_loop(0, NCH_S, body, 0)
    plsc.subcore_barrier()
    pltpu.sync_copy(acc.at[pl.ds(s * CPR, CPR)],
                    agg_hbm.at[pl.ds(nbase + s * CPR, CPR)])


def _ln4(u, g, beta, bdo):
    mu = jnp.dot(u, bdo, preferred_element_type=jnp.float32)
    d = u - mu
    var = jnp.dot(d * d, bdo, preferred_element_type=jnp.float32)
    return d * lax.rsqrt(var + 1e-5) * g + beta


def _full(a):
    return pl.BlockSpec(a.shape, lambda i: tuple(0 for _ in a.shape))


def _rows(block, width):
    return pl.BlockSpec((block, width), lambda i: (i, 0))


def _enc_body(x_ref, mean_ref, std_ref, W1, b1, W2, b2, g, beta, bdo,
              out_ref):
    xn = (x_ref[...] - mean_ref[...]) / std_ref[...]
    t = jnp.maximum(jnp.dot(xn, W1[...], preferred_element_type=jnp.float32)
                    + b1[...], 0.0)
    u = jnp.dot(t, W2[...], preferred_element_type=jnp.float32) + b2[...]
    out_ref[...] = _ln4(u, g[...], beta[...], bdo[...])


def _tc_encoder(arr, mean, std, w, bdo, block):
    n = arr.shape[0]
    args = (arr, mean, std, w["W1"], w["b1"], w["W2"], w["b2"], w["g"],
            w["beta"], bdo)
    return pl.pallas_call(
        _enc_body,
        grid=(n // block,),
        in_specs=[_rows(block, arr.shape[1])] + [_full(a) for a in args[1:]],
        out_specs=_rows(block, 128),
        out_shape=jax.ShapeDtypeStruct((n, 128), jnp.float32),
    )(*args)


def _edge_body(ga, gb, e, W1a, W1b, W1c, b1, W2, b2, g, beta, bdo, out_ref):
    t = (jnp.dot(ga[...], W1a[...], preferred_element_type=jnp.float32)
         + jnp.dot(gb[...], W1b[...], preferred_element_type=jnp.float32)
         + jnp.dot(e[...], W1c[...], preferred_element_type=jnp.float32)
         + b1[...])
    t = jnp.maximum(t, 0.0)
    u = jnp.dot(t, W2[...], preferred_element_type=jnp.float32) + b2[...]
    out_ref[...] = _ln4(u, g[...], beta[...], bdo[...]) + e[...]


def _tc_edge(ga, gb, e, w, bdo):
    args = (ga, gb, e, w["W1a"], w["W1b"], w["W1c"], w["b1"], w["W2"],
            w["b2"], w["g"], w["beta"], bdo)
    return pl.pallas_call(
        _edge_body,
        grid=(E4 // BE4,),
        in_specs=[_rows(BE4, 128)] * 3 + [_full(a) for a in args[3:]],
        out_specs=_rows(BE4, 128),
        out_shape=jax.ShapeDtypeStruct((E4, 128), jnp.float32),
    )(*args)


def _node_body(h, agg, W1a, W1b, b1, W2, b2, g, beta, bdo, out_ref):
    t = (jnp.dot(h[...], W1a[...], preferred_element_type=jnp.float32)
         + jnp.dot(agg[...], W1b[...], preferred_element_type=jnp.float32)
         + b1[...])
    t = jnp.maximum(t, 0.0)
    u = jnp.dot(t, W2[...], preferred_element_type=jnp.float32) + b2[...]
    out_ref[...] = h[...] + _ln4(u, g[...], beta[...], bdo[...])


def _tc_node(h, agg, w, bdo):
    args = (h, agg, w["W1a"], w["W1b"], w["b1"], w["W2"], w["b2"],
            w["g"], w["beta"], bdo)
    return pl.pallas_call(
        _node_body,
        grid=(N4 // BN4,),
        in_specs=[_rows(BN4, 128)] * 2 + [_full(a) for a in args[2:]],
        out_specs=_rows(BN4, 128),
        out_shape=jax.ShapeDtypeStruct((N4, 128), jnp.float32),
    )(*args)


def _dec_body(h, W1, b1, W2, b2, out_ref):
    t = jnp.maximum(jnp.dot(h[...], W1[...],
                            preferred_element_type=jnp.float32) + b1[...], 0.0)
    out_ref[...] = jnp.dot(t, W2[...],
                           preferred_element_type=jnp.float32) + b2[...]


def _tc_dec(h, d):
    args = (h, d["W1"], d["b1"], d["W2"], d["b2"])
    return pl.pallas_call(
        _dec_body,
        grid=(N4 // BN4,),
        in_specs=[_rows(BN4, 128)] + [_full(a) for a in args[1:]],
        out_specs=pl.BlockSpec((BN4, 8), lambda i: (i, 0)),
        out_shape=jax.ShapeDtypeStruct((N4, 8), jnp.float32),
    )(*args)


def _bd4(W):
    return jax.scipy.linalg.block_diag(W, W, W, W)


def _t4(v):
    return jnp.tile(v.reshape(1, -1), (1, 4))


def _prep_mlp4(pr):
    return {"W1": _bd4(pr["W1"]), "b1": _t4(pr["b1"]),
            "W2": _bd4(pr["W2"]), "b2": _t4(pr["b2"]),
            "g": _t4(pr["g"]), "beta": _t4(pr["beta"])}


def kernel(x, edge_index, edge_attr, p, mean_vec_x, std_vec_x,
           mean_vec_edge, std_vec_edge, params):
    pad = E_PAD - E
    src = edge_index[0]
    dst = edge_index[1]
    src_g2 = jnp.concatenate([src, jnp.zeros((pad,), jnp.int32)]
                             ).reshape(E_PAD // 128, 128)
    dst_g2 = jnp.concatenate([dst, jnp.zeros((pad,), jnp.int32)]
                             ).reshape(E_PAD // 128, 128)
    src_s2 = jnp.concatenate([src, jnp.full((pad,), N, jnp.int32)]
                             ).reshape(E_PAD // 128, 128)
    ea4 = jnp.concatenate(
        [edge_attr, jnp.zeros((pad, edge_attr.shape[1]), jnp.float32)]
    ).reshape(E4, 4 * edge_attr.shape[1])
    x4 = x.reshape(N4, 4 * x.shape[1])
    zeros_tile = jnp.zeros((ZROWS, HID), jnp.float32)
    bdo = _bd4(jnp.full((HID, HID), 1.0 / HID, jnp.float32))

    h4 = _tc_encoder(x4, _t4(mean_vec_x), _t4(std_vec_x),
                     _prep_mlp4(params["node_enc"]), bdo, BN4)
    e4 = _tc_encoder(ea4, _t4(mean_vec_edge), _t4(std_vec_edge),
                     _prep_mlp4(params["edge_enc"]), bdo, BE4)

    for lp in params["layers"]:
        em = lp["edge_mlp"]
        ew = {"W1a": _bd4(em["W1"][:HID]), "W1b": _bd4(em["W1"][HID:2 * HID]),
              "W1c": _bd4(em["W1"][2 * HID:]), "b1": _t4(em["b1"]),
              "W2": _bd4(em["W2"]), "b2": _t4(em["b2"]),
              "g": _t4(em["g"]), "beta": _t4(em["beta"])}
        nm = lp["node_mlp"]
        nw = {"W1a": _bd4(nm["W1"][:HID]), "W1b": _bd4(nm["W1"][HID:]),
              "b1": _t4(nm["b1"]), "W2": _bd4(nm["W2"]), "b2": _t4(nm["b2"]),
              "g": _t4(nm["g"]), "beta": _t4(nm["beta"])}
        ga, gb = _sc_gather(h4.reshape(N, HID), dst_g2, src_g2)
        upd4 = _tc_edge(ga.reshape(E4, 128), gb.reshape(E4, 128), e4, ew, bdo)
        agg = _sc_scatter(upd4.reshape(E_PAD, HID), src_s2, zeros_tile)
        h4 = _tc_node(h4, agg.reshape(N4, 128), nw, bdo)
        e4 = upd4

    d = params["dec"]
    out4 = _tc_dec(h4, {"W1": _bd4(d["W1"]), "b1": _t4(d["b1"]),
                        "W2": _bd4(d["W2"]), "b2": _t4(d["b2"])})
    return out4.reshape(N, 2)


# speedup vs baseline: 3.6884x; 1.0213x over previous
import functools

import jax
import jax.numpy as jnp
from jax import lax
from jax.experimental import pallas as pl
from jax.experimental.pallas import tpu as pltpu
from jax.experimental.pallas import tpu_sc as plsc

N = 100000
E = 1600000
HID = 32

NC = 2
NS = 16
NW = NC * NS
CH = 896
EPW = 50176
E_PAD = NW * EPW
NCH_G = EPW // CH
CHS = 256
PER_TILE = E_PAD // NS
NCH_S = PER_TILE // CHS
HALF = N // NC
ZROWS = 3128
ACC_ROWS = NS * ZROWS
CPR = N // NC // NS

E4 = E_PAD // 4
N4 = N // 4
BE4 = 2048
BN4 = 5000

_mesh = plsc.VectorSubcoreMesh(core_axis_name="c", subcore_axis_name="s")


@functools.partial(
    pl.kernel,
    mesh=_mesh,
    compiler_params=pltpu.CompilerParams(use_tc_tiling_on_sc=False),
    out_type=(
        jax.ShapeDtypeStruct((E_PAD, HID), jnp.float32),
        jax.ShapeDtypeStruct((E_PAD, HID), jnp.float32),
    ),
    scratch_types=[
        pltpu.VMEM((CH,), jnp.int32),
        pltpu.VMEM((CH,), jnp.int32),
        pltpu.VMEM((CH,), jnp.int32),
        pltpu.VMEM((CH,), jnp.int32),
        pltpu.VMEM((CH, HID), jnp.float32),
        pltpu.VMEM((CH, HID), jnp.float32),
        pltpu.VMEM((CH, HID), jnp.float32),
        pltpu.VMEM((CH, HID), jnp.float32),
        pltpu.SemaphoreType.DMA,
        pltpu.SemaphoreType.DMA,
        pltpu.SemaphoreType.DMA,
        pltpu.SemaphoreType.DMA,
        pltpu.SemaphoreType.DMA,
        pltpu.SemaphoreType.DMA,
    ],
)
def _sc_gather(h_hbm, dst1_hbm, src1_hbm, ga_hbm, gb_hbm,
               idxd0, idxs0, idxd1, idxs1, ga0, gb0, ga1, gb1,
               semi0, semi1, semg0, semg1, sems0, sems1):
    c = lax.axis_index("c")
    s = lax.axis_index("s")
    wid = s * NC + c
    base = wid * EPW
    idxd = [idxd0, idxd1]
    idxs = [idxs0, idxs1]
    ga_v = [ga0, ga1]
    gb_v = [gb0, gb1]
    semi = [semi0, semi1]
    semg = [semg0, semg1]
    sems = [sems0, sems1]

    def fire_idx(i, b):
        e0 = base + i * CH
        pltpu.async_copy(dst1_hbm.at[pl.ds(e0, CH)], idxd[b], semi[b])
        pltpu.async_copy(src1_hbm.at[pl.ds(e0, CH)], idxs[b], semi[b])

    fire_idx(0, 0)

    def chunk(i, b):
        @pl.when(i >= 2)
        def _():
            pltpu.make_async_copy(ga_v[b], ga_hbm.at[pl.ds(0, CH)],
                                  sems[b]).wait()
            pltpu.make_async_copy(gb_v[b], gb_hbm.at[pl.ds(0, CH)],
                                  sems[b]).wait()

        @pl.when(i + 1 < NCH_G)
        def _():
            fire_idx(i + 1, 1 - b)

        pltpu.make_async_copy(dst1_hbm.at[pl.ds(0, CH)], idxd[b],
                              semi[b]).wait()
        pltpu.make_async_copy(src1_hbm.at[pl.ds(0, CH)], idxs[b],
                              semi[b]).wait()
        for j in range(CH // 128):
            pltpu.async_copy(h_hbm.at[idxd[b].at[pl.ds(j * 128, 128)]],
                             ga_v[b].at[pl.ds(j * 128, 128)], semg[b])
        for j in range(CH // 128):
            pltpu.async_copy(h_hbm.at[idxs[b].at[pl.ds(j * 128, 128)]],
                             gb_v[b].at[pl.ds(j * 128, 128)], semg[b])
        pltpu.make_async_copy(h_hbm.at[pl.ds(0, CH)], ga_v[b],
                              semg[b]).wait()
        pltpu.make_async_copy(h_hbm.at[pl.ds(0, CH)], gb_v[b],
                              semg[b]).wait()
        e0 = base + i * CH
        pltpu.async_copy(ga_v[b], ga_hbm.at[pl.ds(e0, CH)], sems[b])
        pltpu.async_copy(gb_v[b], gb_hbm.at[pl.ds(e0, CH)], sems[b])

    def body(ii, carry):
        for b in range(2):
            chunk(ii * 2 + b, b)
        return carry

    lax.fori_loop(0, NCH_G // 2, body, 0)
    for b in range(2):
        pltpu.make_async_copy(ga_v[b], ga_hbm.at[pl.ds(0, CH)],
                              sems[b]).wait()
        pltpu.make_async_copy(gb_v[b], gb_hbm.at[pl.ds(0, CH)],
                              sems[b]).wait()


@functools.partial(
    pl.kernel,
    mesh=_mesh,
    compiler_params=pltpu.CompilerParams(use_tc_tiling_on_sc=False),
    out_type=jax.ShapeDtypeStruct((N, HID), jnp.float32),
    scratch_types=[
        pltpu.VMEM((CHS,), jnp.int32),
        pltpu.VMEM((CHS,), jnp.int32),
        pltpu.VMEM((CHS // 128, 128), jnp.int32),
        pltpu.VMEM((CHS // 128, 128), jnp.int32),
        pltpu.VMEM((CHS, HID), jnp.float32),
        pltpu.VMEM((CHS, HID), jnp.float32),
        pltpu.VMEM_SHARED((ACC_ROWS, HID), jnp.float32),
        pltpu.SemaphoreType.DMA,
        pltpu.SemaphoreType.DMA,
        pltpu.SemaphoreType.DMA,
        pltpu.SemaphoreType.DMA,
    ],
)
def _sc_scatter(upd_hbm, src1_hbm, zeros_hbm, agg_hbm,
                idxs0, idxs1, idxl0, idxl1, rows0, rows1, acc,
                semi0, semi1, semr0, semr1):
    c = lax.axis_index("c")
    s = lax.axis_index("s")
    nbase = c * HALF
    idxs = [idxs0, idxs1]
    idxl = [idxl0, idxl1]
    rows = [rows0, rows1]
    semi = [semi0, semi1]
    semr = [semr0, semr1]
    pltpu.sync_copy(zeros_hbm, acc.at[pl.ds(s * ZROWS, ZROWS)])
    plsc.subcore_barrier()

    t0 = s * PER_TILE

    def fire(i, b):
        e0 = t0 + i * CHS
        pltpu.async_copy(src1_hbm.at[pl.ds(e0, CHS)], idxs[b], semi[b])
        pltpu.async_copy(upd_hbm.at[pl.ds(e0, CHS)], rows[b], semr[b])

    fire(0, 0)

    def chunk(i, b):
        @pl.when(i + 1 < NCH_S)
        def _():
            fire(i + 1, 1 - b)

        pltpu.make_async_copy(src1_hbm.at[pl.ds(0, CHS)], idxs[b],
                              semi[b]).wait()
        for k in range(CHS // 16):
            v = idxs[b][pl.ds(k * 16, 16)]
            loc = v - nbase
            ok = (loc >= 0) & (loc < HALF)
            idxl[b][k // 8, pl.ds((k % 8) * 16, 16)] = jnp.where(ok, loc,
                                                                 HALF)
        pltpu.make_async_copy(upd_hbm.at[pl.ds(0, CHS)], rows[b],
                              semr[b]).wait()
        for j in range(CHS // 128):
            pltpu.sync_copy(rows[b].at[pl.ds(j * 128, 128)],
                            acc.at[idxl[b].at[j]], add=True)

    def body(ii, carry):
        for b in range(2):
            chunk(ii * 2 + b, b)
        return carry

    lax.fori_loop(0, NCH_S // 2, body, 0)
    plsc.subcore_barrier()
    pltpu.sync_copy(acc.at[pl.ds(s * CPR, CPR)],
                    agg_hbm.at[pl.ds(nbase + s * CPR, CPR)])


def _ln4(u, g, beta, bdo):
    mu = jnp.dot(u, bdo, preferred_element_type=jnp.float32)
    d = u - mu
    var = jnp.dot(d * d, bdo, preferred_element_type=jnp.float32)
    return d * lax.rsqrt(var + 1e-5) * g + beta


def _full(a):
    return pl.BlockSpec(a.shape, lambda i: tuple(0 for _ in a.shape))


def _rows(block, width):
    return pl.BlockSpec((block, width), lambda i: (i, 0))


def _enc_body(x_ref, mean_ref, std_ref, W1, b1, W2, b2, g, beta, bdo,
              out_ref):
    xn = (x_ref[...] - mean_ref[...]) / std_ref[...]
    t = jnp.maximum(jnp.dot(xn, W1[...], preferred_element_type=jnp.float32)
                    + b1[...], 0.0)
    u = jnp.dot(t, W2[...], preferred_element_type=jnp.float32) + b2[...]
    out_ref[...] = _ln4(u, g[...], beta[...], bdo[...])


def _tc_encoder(arr, mean, std, w, bdo, block):
    n = arr.shape[0]
    args = (arr, mean, std, w["W1"], w["b1"], w["W2"], w["b2"], w["g"],
            w["beta"], bdo)
    return pl.pallas_call(
        _enc_body,
        grid=(n // block,),
        in_specs=[_rows(block, arr.shape[1])] + [_full(a) for a in args[1:]],
        out_specs=_rows(block, 128),
        out_shape=jax.ShapeDtypeStruct((n, 128), jnp.float32),
    )(*args)


def _edge_body(ga, gb, e, W1a, W1b, W1c, b1, W2, b2, g, beta, bdo, out_ref):
    t = (jnp.dot(ga[...], W1a[...], preferred_element_type=jnp.float32)
         + jnp.dot(gb[...], W1b[...], preferred_element_type=jnp.float32)
         + jnp.dot(e[...], W1c[...], preferred_element_type=jnp.float32)
         + b1[...])
    t = jnp.maximum(t, 0.0)
    u = jnp.dot(t, W2[...], preferred_element_type=jnp.float32) + b2[...]
    out_ref[...] = _ln4(u, g[...], beta[...], bdo[...]) + e[...]


def _tc_edge(ga, gb, e, w, bdo):
    args = (ga, gb, e, w["W1a"], w["W1b"], w["W1c"], w["b1"], w["W2"],
            w["b2"], w["g"], w["beta"], bdo)
    return pl.pallas_call(
        _edge_body,
        grid=(E4 // BE4,),
        in_specs=[_rows(BE4, 128)] * 3 + [_full(a) for a in args[3:]],
        out_specs=_rows(BE4, 128),
        out_shape=jax.ShapeDtypeStruct((E4, 128), jnp.float32),
    )(*args)


def _node_body(h, agg, W1a, W1b, b1, W2, b2, g, beta, bdo, out_ref):
    t = (jnp.dot(h[...], W1a[...], preferred_element_type=jnp.float32)
         + jnp.dot(agg[...], W1b[...], preferred_element_type=jnp.float32)
         + b1[...])
    t = jnp.maximum(t, 0.0)
    u = jnp.dot(t, W2[...], preferred_element_type=jnp.float32) + b2[...]
    out_ref[...] = h[...] + _ln4(u, g[...], beta[...], bdo[...])


def _tc_node(h, agg, w, bdo):
    args = (h, agg, w["W1a"], w["W1b"], w["b1"], w["W2"], w["b2"],
            w["g"], w["beta"], bdo)
    return pl.pallas_call(
        _node_body,
        grid=(N4 // BN4,),
        in_specs=[_rows(BN4, 128)] * 2 + [_full(a) for a in args[2:]],
        out_specs=_rows(BN4, 128),
        out_shape=jax.ShapeDtypeStruct((N4, 128), jnp.float32),
    )(*args)


def _dec_body(h, W1, b1, W2, b2, out_ref):
    t = jnp.maximum(jnp.dot(h[...], W1[...],
                            preferred_element_type=jnp.float32) + b1[...], 0.0)
    out_ref[...] = jnp.dot(t, W2[...],
                           preferred_element_type=jnp.float32) + b2[...]


def _tc_dec(h, d):
    args = (h, d["W1"], d["b1"], d["W2"], d["b2"])
    return pl.pallas_call(
        _dec_body,
        grid=(N4 // BN4,),
        in_specs=[_rows(BN4, 128)] + [_full(a) for a in args[1:]],
        out_specs=pl.BlockSpec((BN4, 8), lambda i: (i, 0)),
        out_shape=jax.ShapeDtypeStruct((N4, 8), jnp.float32),
    )(*args)


def _bd4(W):
    return jax.scipy.linalg.block_diag(W, W, W, W)


def _t4(v):
    return jnp.tile(v.reshape(1, -1), (1, 4))


def _prep_mlp4(pr):
    return {"W1": _bd4(pr["W1"]), "b1": _t4(pr["b1"]),
            "W2": _bd4(pr["W2"]), "b2": _t4(pr["b2"]),
            "g": _t4(pr["g"]), "beta": _t4(pr["beta"])}


def kernel(x, edge_index, edge_attr, p, mean_vec_x, std_vec_x,
           mean_vec_edge, std_vec_edge, params):
    pad = E_PAD - E
    src = edge_index[0]
    dst = edge_index[1]
    src_g1 = jnp.concatenate([src, jnp.zeros((pad,), jnp.int32)])
    dst_g1 = jnp.concatenate([dst, jnp.zeros((pad,), jnp.int32)])
    src_s1 = jnp.concatenate([src, jnp.full((pad,), N, jnp.int32)])
    ea4 = jnp.concatenate(
        [edge_attr, jnp.zeros((pad, edge_attr.shape[1]), jnp.float32)]
    ).reshape(E4, 4 * edge_attr.shape[1])
    x4 = x.reshape(N4, 4 * x.shape[1])
    zeros_tile = jnp.zeros((ZROWS, HID), jnp.float32)
    bdo = _bd4(jnp.full((HID, HID), 1.0 / HID, jnp.float32))

    h4 = _tc_encoder(x4, _t4(mean_vec_x), _t4(std_vec_x),
                     _prep_mlp4(params["node_enc"]), bdo, BN4)
    e4 = _tc_encoder(ea4, _t4(mean_vec_edge), _t4(std_vec_edge),
                     _prep_mlp4(params["edge_enc"]), bdo, BE4)

    for lp in params["layers"]:
        em = lp["edge_mlp"]
        ew = {"W1a": _bd4(em["W1"][:HID]), "W1b": _bd4(em["W1"][HID:2 * HID]),
              "W1c": _bd4(em["W1"][2 * HID:]), "b1": _t4(em["b1"]),
              "W2": _bd4(em["W2"]), "b2": _t4(em["b2"]),
              "g": _t4(em["g"]), "beta": _t4(em["beta"])}
        nm = lp["node_mlp"]
        nw = {"W1a": _bd4(nm["W1"][:HID]), "W1b": _bd4(nm["W1"][HID:]),
              "b1": _t4(nm["b1"]), "W2": _bd4(nm["W2"]), "b2": _t4(nm["b2"]),
              "g": _t4(nm["g"]), "beta": _t4(nm["beta"])}
        ga, gb = _sc_gather(h4.reshape(N, HID), dst_g1, src_g1)
        upd4 = _tc_edge(ga.reshape(E4, 128), gb.reshape(E4, 128), e4, ew, bdo)
        agg = _sc_scatter(upd4.reshape(E_PAD, HID), src_s1, zeros_tile)
        h4 = _tc_node(h4, agg.reshape(N4, 128), nw, bdo)
        e4 = upd4

    d = params["dec"]
    out4 = _tc_dec(h4, {"W1": _bd4(d["W1"]), "b1": _t4(d["b1"]),
                        "W2": _bd4(d["W2"]), "b2": _t4(d["b2"])})
    return out4.reshape(N, 2)
